# baseline (speedup 1.0000x reference)
import functools

import jax
import jax.numpy as jnp
from jax import lax
from jax.experimental import pallas as pl
from jax.experimental.pallas import tpu as pltpu
from jax.experimental.pallas import tpu_sc as plsc

N_NODES = 100000
MAX_DEGREE = 64
BATCH = 16384
N_SAMPLES = 25

NC = 2
NS = 16
NW = NC * NS
BPW = BATCH // NW
CHUNK = 128
NCHUNK = BPW // CHUNK
OPW = BPW * N_SAMPLES
NVEC = OPW // 16

_mesh = plsc.VectorSubcoreMesh(core_axis_name="c", subcore_axis_name="s")


@functools.partial(
    pl.kernel,
    mesh=_mesh,
    compiler_params=pltpu.CompilerParams(
        needs_layout_passes=False, use_tc_tiling_on_sc=False),
    out_type=(
        jax.ShapeDtypeStruct((BATCH * N_SAMPLES,), jnp.int32),
        jax.ShapeDtypeStruct((BATCH * N_SAMPLES,), jnp.int32),
    ),
    scratch_types=[
        pltpu.VMEM((NCHUNK, CHUNK), jnp.int32),
        pltpu.VMEM((NCHUNK, CHUNK), jnp.int32),
        pltpu.VMEM((BPW, MAX_DEGREE), jnp.int32),
        pltpu.VMEM((BPW, MAX_DEGREE), jnp.int32),
        pltpu.VMEM((OPW,), jnp.int32),
        pltpu.VMEM((OPW,), jnp.int32),
        pltpu.VMEM((OPW,), jnp.int32),
        pltpu.VMEM((OPW,), jnp.int32),
        pltpu.SemaphoreType.DMA,
    ],
)
def _sample_neighbors(ids_hbm, prow_hbm, pcol_hbm, tbl_hbm,
                      out_info_hbm, out_ans_hbm,
                      idx_v, aidx_v, rows_info, rows_ans, prow_v, pcol_v,
                      out_info_v, out_ans_v, sem):
    wid = lax.axis_index("s") * NC + lax.axis_index("c")

    pltpu.sync_copy(ids_hbm.at[wid], idx_v)
    for c in range(NCHUNK):
        for k in range(CHUNK // 16):
            s = pl.ds(k * 16, 16)
            aidx_v[c, s] = idx_v[c, s] + N_NODES

    copies = []
    for c in range(NCHUNK):
        dst = pl.ds(c * CHUNK, CHUNK)
        copies.append(
            pltpu.async_copy(tbl_hbm.at[idx_v.at[c]], rows_info.at[dst], sem))
        copies.append(
            pltpu.async_copy(tbl_hbm.at[aidx_v.at[c]], rows_ans.at[dst], sem))
    pltpu.sync_copy(prow_hbm, prow_v)
    pltpu.sync_copy(pcol_hbm, pcol_v)
    for cp in copies:
        cp.wait()

    def body(i, carry):
        s = pl.ds(i * 16, 16)
        r = prow_v[s]
        c = pcol_v[s]
        out_info_v[s] = plsc.load_gather(rows_info, [r, c])
        out_ans_v[s] = plsc.load_gather(rows_ans, [r, c])
        return carry

    lax.fori_loop(0, NVEC, body, 0)

    out = pl.ds(wid * OPW, OPW)
    pltpu.sync_copy(out_info_v, out_info_hbm.at[out])
    pltpu.sync_copy(out_ans_v, out_ans_hbm.at[out])


def kernel(ids, num_samples, adj_info, adj_answer):
    perm = jax.random.permutation(jax.random.key(42), MAX_DEGREE)
    start = jnp.asarray(num_samples, jnp.int32) - N_SAMPLES
    cols = lax.dynamic_slice(perm, (start,), (N_SAMPLES,)).astype(jnp.int32)

    prow = jnp.repeat(jnp.arange(BPW, dtype=jnp.int32), N_SAMPLES)
    pcol = jnp.tile(cols, BPW)

    tbl = jnp.concatenate([adj_info, adj_answer], axis=0)
    ids3 = ids.astype(jnp.int32).reshape(NW, NCHUNK, CHUNK)
    o_info, o_ans = _sample_neighbors(ids3, prow, pcol, tbl)
    return (o_info.reshape(BATCH, N_SAMPLES), o_ans.reshape(BATCH, N_SAMPLES))


# speedup vs baseline: 1.4639x; 1.4639x over previous
import functools

import jax
import jax.numpy as jnp
from jax import lax
from jax.experimental import pallas as pl
from jax.experimental.pallas import tpu as pltpu
from jax.experimental.pallas import tpu_sc as plsc

N_NODES = 100000
MAX_DEGREE = 64
BATCH = 16384
N_SAMPLES = 25

NC = 2
NS = 16
NW = NC * NS
BPW = BATCH // NW
CHUNK = 128
NCHUNK = BPW // CHUNK
PAIRW = 2 * MAX_DEGREE
OPW = BPW * N_SAMPLES
NVEC = OPW // 16

_mesh = plsc.VectorSubcoreMesh(core_axis_name="c", subcore_axis_name="s")


@functools.partial(
    pl.kernel,
    mesh=_mesh,
    compiler_params=pltpu.CompilerParams(
        needs_layout_passes=False, use_tc_tiling_on_sc=True),
    out_type=jax.ShapeDtypeStruct((BATCH * N_SAMPLES,), jnp.int32),
    scratch_types=[
        pltpu.VMEM((BPW,), jnp.int32),
        pltpu.VMEM((8, CHUNK), jnp.int32),
        pltpu.VMEM((BPW,), jnp.int32),
        pltpu.VMEM((BPW, PAIRW), jnp.int32),
        pltpu.VMEM((OPW,), jnp.int32),
        pltpu.VMEM((OPW,), jnp.int32),
        pltpu.VMEM((OPW,), jnp.int32),
        pltpu.SemaphoreType.DMA,
    ],
)
def _sample_one_table(ids_hbm, prow_hbm, pcol_hbm, tbl_hbm, out_hbm,
                      idx_v, gidx_v, par_v, rows_v, prow_v, pcol_v,
                      out_v, sem):
    wid = lax.axis_index("s") * NC + lax.axis_index("c")

    pltpu.sync_copy(ids_hbm.at[pl.ds(wid * BPW, BPW)], idx_v)
    for c in range(NCHUNK):
        for k in range(CHUNK // 16):
            s = pl.ds(c * CHUNK + k * 16, 16)
            v = idx_v[s]
            gidx_v[c, pl.ds(k * 16, 16)] = v >> 1
            par_v[s] = (v & 1) << 6

    copies = []
    for c in range(NCHUNK):
        dst = pl.ds(c * CHUNK, CHUNK)
        copies.append(
            pltpu.async_copy(tbl_hbm.at[gidx_v.at[c]], rows_v.at[dst], sem))
    pltpu.sync_copy(prow_hbm, prow_v)
    pltpu.sync_copy(pcol_hbm, pcol_v)
    for cp in copies:
        cp.wait()

    def body(i, carry):
        s = pl.ds(i * 16, 16)
        r = prow_v[s]
        c = pcol_v[s] + plsc.load_gather(par_v, [r])
        out_v[s] = plsc.load_gather(rows_v, [r, c])
        return carry

    lax.fori_loop(0, NVEC, body, 0)

    pltpu.sync_copy(out_v, out_hbm.at[pl.ds(wid * OPW, OPW)])


def kernel(ids, num_samples, adj_info, adj_answer):
    perm = jax.random.permutation(jax.random.key(42), MAX_DEGREE)
    start = jnp.asarray(num_samples, jnp.int32) - N_SAMPLES
    cols = lax.dynamic_slice(perm, (start,), (N_SAMPLES,)).astype(jnp.int32)

    prow = jnp.repeat(jnp.arange(BPW, dtype=jnp.int32), N_SAMPLES)
    pcol = jnp.tile(cols, BPW)

    ids32 = ids.astype(jnp.int32)
    o_info = _sample_one_table(
        ids32, prow, pcol, adj_info.reshape(N_NODES // 2, PAIRW))
    o_ans = _sample_one_table(
        ids32, prow, pcol, adj_answer.reshape(N_NODES // 2, PAIRW))
    return (o_info.reshape(BATCH, N_SAMPLES), o_ans.reshape(BATCH, N_SAMPLES))


# speedup vs baseline: 1.6138x; 1.1024x over previous
import functools

import jax
import jax.numpy as jnp
from jax import lax
from jax.experimental import pallas as pl
from jax.experimental.pallas import tpu as pltpu
from jax.experimental.pallas import tpu_sc as plsc

N_NODES = 100000
MAX_DEGREE = 64
BATCH = 16384
N_SAMPLES = 25

NC = 2
NS = 16
NW = NC * NS
BPW = BATCH // NW
CHUNK = 128
NCHUNK = BPW // CHUNK
PAIRW = 2 * MAX_DEGREE
OPW = BPW * N_SAMPLES
VPC = BPW // 16
UNROLL = 4

_mesh = plsc.VectorSubcoreMesh(core_axis_name="c", subcore_axis_name="s")


@functools.partial(
    pl.kernel,
    mesh=_mesh,
    compiler_params=pltpu.CompilerParams(
        needs_layout_passes=False, use_tc_tiling_on_sc=True),
    out_type=jax.ShapeDtypeStruct((BATCH * N_SAMPLES,), jnp.int32),
    scratch_types=[
        pltpu.VMEM((BPW,), jnp.int32),
        pltpu.VMEM((8, CHUNK), jnp.int32),
        pltpu.VMEM((BPW,), jnp.int32),
        pltpu.VMEM((BPW, PAIRW), jnp.int32),
        pltpu.VMEM((OPW,), jnp.int32),
        pltpu.VMEM((32,), jnp.int32),
        pltpu.SemaphoreType.DMA,
        pltpu.SemaphoreType.DMA,
    ],
)
def _sample_one_table(ids_hbm, cols_hbm, tbl_hbm, out_hbm,
                      idx_v, gidx_v, par_v, rows_v, out_v, cols_v,
                      sem, sem_out):
    wid = lax.axis_index("s") * NC + lax.axis_index("c")

    pltpu.sync_copy(cols_hbm, cols_v)
    pltpu.sync_copy(ids_hbm.at[pl.ds(wid * BPW, BPW)], idx_v)
    for c in range(NCHUNK):
        for k in range(CHUNK // 16):
            s = pl.ds(c * CHUNK + k * 16, 16)
            v = idx_v[s]
            gidx_v[c, pl.ds(k * 16, 16)] = v >> 1
            par_v[s] = (v & 1) << 6

    copies = []
    for c in range(NCHUNK):
        dst = pl.ds(c * CHUNK, CHUNK)
        copies.append(
            pltpu.async_copy(tbl_hbm.at[gidx_v.at[c]], rows_v.at[dst], sem))
    for cp in copies:
        cp.wait()

    lanes = lax.iota(jnp.int32, 16)
    cols_lo = cols_v[pl.ds(0, 16)]
    cols_hi = cols_v[pl.ds(16, 16)]

    for j in range(N_SAMPLES):
        cj = cols_lo[j] if j < 16 else cols_hi[j - 16]
        obase = j * BPW

        def vec_body(i, carry2, cj=cj, obase=obase):
            for u in range(UNROLL):
                s = pl.ds(i * (UNROLL * 16) + u * 16, 16)
                r = lanes + i * (UNROLL * 16) + u * 16
                c = par_v[s] + cj
                out_v[pl.ds(obase + i * (UNROLL * 16) + u * 16, 16)] = (
                    plsc.load_gather(rows_v, [r, c]))
            return carry2

        lax.fori_loop(0, VPC // UNROLL, vec_body, 0)
        pltpu.async_copy(
            out_v.at[pl.ds(obase, BPW)],
            out_hbm.at[pl.ds(j * BATCH + wid * BPW, BPW)], sem_out)

    def drain(j, carry):
        pltpu.make_async_copy(
            out_v.at[pl.ds(0, BPW)],
            out_hbm.at[pl.ds(wid * BPW, BPW)], sem_out).wait()
        return carry

    lax.fori_loop(0, N_SAMPLES, drain, 0)


def kernel(ids, num_samples, adj_info, adj_answer):
    perm = jax.random.permutation(jax.random.key(42), MAX_DEGREE)
    start = jnp.asarray(num_samples, jnp.int32) - N_SAMPLES
    cols = lax.dynamic_slice(perm, (start,), (N_SAMPLES,)).astype(jnp.int32)
    cols32 = jnp.concatenate([cols, jnp.zeros((32 - N_SAMPLES,), jnp.int32)])

    ids32 = ids.astype(jnp.int32)
    o_info = _sample_one_table(
        ids32, cols32, adj_info.reshape(N_NODES // 2, PAIRW))
    o_ans = _sample_one_table(
        ids32, cols32, adj_answer.reshape(N_NODES // 2, PAIRW))
    return (
        o_info.reshape(N_SAMPLES, BATCH).T,
        o_ans.reshape(N_SAMPLES, BATCH).T,
    )


# speedup vs baseline: 1.6440x; 1.0187x over previous
import functools

import jax
import jax.numpy as jnp
from jax import lax
from jax.experimental import pallas as pl
from jax.experimental.pallas import tpu as pltpu
from jax.experimental.pallas import tpu_sc as plsc

N_NODES = 100000
MAX_DEGREE = 64
BATCH = 16384
N_SAMPLES = 25

NC = 2
NS = 16
NW = NC * NS
BPW = BATCH // NW
CHUNK = 128
NCHUNK = BPW // CHUNK
PAIRW = 2 * MAX_DEGREE
OPW = BPW * N_SAMPLES
VPC = BPW // 16
UNROLL = 4

_mesh = plsc.VectorSubcoreMesh(core_axis_name="c", subcore_axis_name="s")


@functools.partial(
    pl.kernel,
    mesh=_mesh,
    compiler_params=pltpu.CompilerParams(
        needs_layout_passes=False, use_tc_tiling_on_sc=True),
    out_type=jax.ShapeDtypeStruct((N_SAMPLES, BATCH), jnp.int32),
    scratch_types=[
        pltpu.VMEM((BPW,), jnp.int32),
        pltpu.VMEM((8, CHUNK), jnp.int32),
        pltpu.VMEM((BPW,), jnp.int32),
        pltpu.VMEM((BPW, PAIRW), jnp.int32),
        pltpu.VMEM((32, BPW), jnp.int32),
        pltpu.VMEM((32,), jnp.int32),
        pltpu.SemaphoreType.DMA,
        pltpu.SemaphoreType.DMA,
    ],
)
def _sample_one_table(ids_hbm, cols_hbm, tbl_hbm, out_hbm,
                      idx_v, gidx_v, par_v, rows_v, out_v, cols_v,
                      sem, sem_out):
    wid = lax.axis_index("s") * NC + lax.axis_index("c")

    pltpu.sync_copy(cols_hbm, cols_v)
    pltpu.sync_copy(ids_hbm.at[pl.ds(wid * BPW, BPW)], idx_v)
    for c in range(NCHUNK):
        for k in range(CHUNK // 16):
            s = pl.ds(c * CHUNK + k * 16, 16)
            v = idx_v[s]
            gidx_v[c, pl.ds(k * 16, 16)] = v >> 1
            par_v[s] = (v & 1) << 6

    copies = []
    for c in range(NCHUNK):
        dst = pl.ds(c * CHUNK, CHUNK)
        copies.append(
            pltpu.async_copy(tbl_hbm.at[gidx_v.at[c]], rows_v.at[dst], sem))
    for cp in copies:
        cp.wait()

    lanes = lax.iota(jnp.int32, 16)
    cols_lo = cols_v[pl.ds(0, 16)]
    cols_hi = cols_v[pl.ds(16, 16)]

    for j in range(N_SAMPLES):
        cj = cols_lo[j] if j < 16 else cols_hi[j - 16]

        def vec_body(i, carry2, cj=cj, j=j):
            for u in range(UNROLL):
                s = pl.ds(i * (UNROLL * 16) + u * 16, 16)
                r = lanes + i * (UNROLL * 16) + u * 16
                c = par_v[s] + cj
                out_v[j, s] = plsc.load_gather(rows_v, [r, c])
            return carry2

        lax.fori_loop(0, VPC // UNROLL, vec_body, 0)
        pltpu.async_copy(
            out_v.at[pl.ds(j, 1)],
            out_hbm.at[pl.ds(j, 1), pl.ds(wid * BPW, BPW)], sem_out)

    def drain(j, carry):
        pltpu.make_async_copy(
            out_v.at[pl.ds(0, 1)],
            out_hbm.at[pl.ds(0, 1), pl.ds(wid * BPW, BPW)], sem_out).wait()
        return carry

    lax.fori_loop(0, N_SAMPLES, drain, 0)


def kernel(ids, num_samples, adj_info, adj_answer):
    perm = jax.random.permutation(jax.random.key(42), MAX_DEGREE)
    start = jnp.asarray(num_samples, jnp.int32) - N_SAMPLES
    cols = lax.dynamic_slice(perm, (start,), (N_SAMPLES,)).astype(jnp.int32)
    cols32 = jnp.concatenate([cols, jnp.zeros((32 - N_SAMPLES,), jnp.int32)])

    ids32 = ids.astype(jnp.int32)
    o_info = _sample_one_table(
        ids32, cols32, adj_info.reshape(N_NODES // 2, PAIRW))
    o_ans = _sample_one_table(
        ids32, cols32, adj_answer.reshape(N_NODES // 2, PAIRW))
    return (o_info.T, o_ans.T)
